# baseline (speedup 1.0000x reference)
import functools

import jax
import jax.numpy as jnp
from jax import lax
from jax.experimental import pallas as pl
from jax.experimental.pallas import tpu as pltpu
from jax.experimental.pallas import tpu_sc as plsc

_L2 = 1e-05
_LANES = 16
_NC, _NS = 2, 16
_NW = _NC * _NS
_CHUNK = 128


def _concat_tc(a, bb, rows_per_block):
    v, d = a.shape

    def body(a_ref, b_ref, o_ref):
        o_ref[...] = jnp.concatenate((a_ref[...], b_ref[...]), axis=-1)

    return pl.pallas_call(
        body,
        grid=(v // rows_per_block,),
        in_specs=[
            pl.BlockSpec((rows_per_block, d), lambda i: (i, 0)),
            pl.BlockSpec((rows_per_block, d), lambda i: (i, 0)),
        ],
        out_specs=pl.BlockSpec((rows_per_block, 2 * d), lambda i: (i, 0)),
        out_shape=jax.ShapeDtypeStruct((v, 2 * d), jnp.float32),
    )(a, bb)


def _combine_sc(qf, qrf, pidf, qq, rr, dflat):
    n = qf.shape[0]
    d2 = qq.shape[1]
    d = d2 // 2
    per_w = n // _NW
    steps = per_w // _CHUNK
    nchunk = d // _LANES
    assert steps % 2 == 0
    mesh = plsc.VectorSubcoreMesh(core_axis_name="c", subcore_axis_name="s")

    @functools.partial(
        pl.kernel,
        mesh=mesh,
        compiler_params=pltpu.CompilerParams(needs_layout_passes=False),
        out_type=[
            jax.ShapeDtypeStruct((n, d), jnp.float32),
            jax.ShapeDtypeStruct((n, d), jnp.float32),
            jax.ShapeDtypeStruct((_NW, _LANES), jnp.float32),
        ],
        scratch_types=[
            pltpu.VMEM((2, _CHUNK), jnp.int32),
            pltpu.VMEM((2, _CHUNK), jnp.int32),
            pltpu.VMEM((2, _CHUNK), jnp.int32),
            pltpu.VMEM((2, _CHUNK, d2), jnp.float32),
            pltpu.VMEM((2, _CHUNK, d2), jnp.float32),
            pltpu.VMEM((2, _CHUNK), jnp.float32),
            pltpu.VMEM((_CHUNK, d), jnp.float32),
            pltpu.VMEM((_CHUNK, d), jnp.float32),
            pltpu.VMEM((_LANES,), jnp.float32),
            pltpu.SemaphoreType.DMA,
            pltpu.SemaphoreType.DMA,
            pltpu.SemaphoreType.DMA,
        ],
    )
    def k(qf_hbm, qrf_hbm, pidf_hbm, qq_t, rr_t, df_t,
          x_hbm, y_hbm, part_hbm,
          iq_v, iqr_v, ipid_v, tq_v, tr_v, dc_v, xo_v, yo_v, acc_v,
          gsem0, gsem1, ssem):
        wid = lax.axis_index("s") * _NC + lax.axis_index("c")
        gsem = (gsem0, gsem1)
        acc_v[...] = jnp.zeros((_LANES,), jnp.float32)

        def fire(b, step):
            base = wid * per_w + step * _CHUNK
            pltpu.sync_copy(qf_hbm.at[pl.ds(base, _CHUNK)], iq_v.at[b])
            pltpu.sync_copy(qrf_hbm.at[pl.ds(base, _CHUNK)], iqr_v.at[b])
            pltpu.sync_copy(pidf_hbm.at[pl.ds(base, _CHUNK)], ipid_v.at[b])
            pltpu.async_copy(qq_t.at[iq_v.at[b]], tq_v.at[b], gsem[b])
            pltpu.async_copy(rr_t.at[iqr_v.at[b]], tr_v.at[b], gsem[b])
            pltpu.async_copy(df_t.at[ipid_v.at[b]], dc_v.at[b], gsem[b])

        def wait_g(b):
            pltpu.make_async_copy(qq_t.at[iq_v.at[b]], tq_v.at[b], gsem[b]).wait()
            pltpu.make_async_copy(rr_t.at[iqr_v.at[b]], tr_v.at[b], gsem[b]).wait()
            pltpu.make_async_copy(df_t.at[ipid_v.at[b]], dc_v.at[b], gsem[b]).wait()

        def compute(b):
            @pl.loop(0, _CHUNK // _LANES)
            def _(g):
                dg = dc_v[b, pl.ds(g * _LANES, _LANES)]
                acc_v[...] += dg * dg

            @pl.loop(0, _CHUNK)
            def _(i):
                dv = plsc.load_gather(
                    dc_v.at[b], [jnp.full((_LANES,), i, jnp.int32)])
                for cc in range(nchunk):
                    sl = pl.ds(cc * _LANES, _LANES)
                    sh = pl.ds(d + cc * _LANES, _LANES)
                    xo_v[i, sl] = tq_v[b, i, sl] + dv * tq_v[b, i, sh]
                    yo_v[i, sl] = tr_v[b, i, sl] + dv * tr_v[b, i, sh]

        def store(step):
            base = wid * per_w + step * _CHUNK
            pltpu.async_copy(xo_v, x_hbm.at[pl.ds(base, _CHUNK)], ssem)
            pltpu.async_copy(yo_v, y_hbm.at[pl.ds(base, _CHUNK)], ssem)

        def wait_s():
            pltpu.make_async_copy(xo_v, x_hbm.at[pl.ds(0, _CHUNK)], ssem).wait()
            pltpu.make_async_copy(yo_v, y_hbm.at[pl.ds(0, _CHUNK)], ssem).wait()

        fire(0, 0)
        wait_g(0)
        fire(1, 1)
        compute(0)
        store(0)

        @pl.loop(0, (steps - 2) // 2)
        def _(p):
            s1 = 2 * p + 1
            s2 = s1 + 1
            wait_g(1)
            fire(0, s2)
            wait_s()
            compute(1)
            store(s1)
            wait_g(0)

            @pl.when(s2 + 1 < steps)
            def _():
                fire(1, s2 + 1)

            wait_s()
            compute(0)
            store(s2)

        wait_g(1)
        wait_s()
        compute(1)
        store(steps - 1)
        wait_s()
        pltpu.sync_copy(acc_v, part_hbm.at[wid])

    return k(qf, qrf, pidf, qq, rr, dflat)


def _xpose_tc(xf, yf, b, l):
    d = xf.shape[1]
    bb, lq = 512, 4
    lp = l // 2
    lblk = lp // lq

    def body(x_ref, y_ref, xo_ref, yo_ref):
        for j in range(lq):
            xo_ref[0, j] = x_ref[:, 0, j, :].T.reshape(2, d, bb)
            yo_ref[0, j] = y_ref[:, 0, j, :].T.reshape(2, d, bb)

    return pl.pallas_call(
        body,
        grid=(b // bb, lblk),
        in_specs=[
            pl.BlockSpec((bb, 1, lq, 2 * d), lambda i, j: (i, j, 0, 0)),
            pl.BlockSpec((bb, 1, lq, 2 * d), lambda i, j: (i, j, 0, 0)),
        ],
        out_specs=[
            pl.BlockSpec((1, lq, 2, d, bb), lambda i, j: (j, 0, 0, 0, i)),
            pl.BlockSpec((1, lq, 2, d, bb), lambda i, j: (j, 0, 0, 0, i)),
        ],
        out_shape=[
            jax.ShapeDtypeStruct((lblk, lq, 2, d, b), jnp.float32),
            jax.ShapeDtypeStruct((lblk, lq, 2, d, b), jnp.float32),
        ],
    )(xf.reshape(b, lblk, lq, 2 * d), yf.reshape(b, lblk, lq, 2 * d))


def _loss_tc(partials):
    def body(p_ref, o_ref):
        o_ref[0, 0] = jnp.sum(p_ref[...]) * jnp.float32(_L2)

    return pl.pallas_call(
        body,
        out_shape=jax.ShapeDtypeStruct((1, 1), jnp.float32),
        out_specs=pl.BlockSpec(memory_space=pltpu.SMEM),
    )(partials)


def kernel(q, qr, pid, q_emb, q_emb_diff, qr_emb, qr_emb_diff, diff_emb):
    b, l = q.shape
    d = q_emb.shape[1]
    qf = q.reshape(-1).astype(jnp.int32)
    qrf = qr.reshape(-1).astype(jnp.int32)
    pidf = pid.reshape(-1).astype(jnp.int32)
    qq = jnp.concatenate([q_emb, q_emb_diff], axis=1)
    rr = jnp.concatenate([qr_emb, qr_emb_diff], axis=1)
    dflat = diff_emb.reshape(-1)
    x, y, parts = _combine_sc(qf, qrf, pidf, qq, rr, dflat)
    loss = _loss_tc(parts)[0, 0]
    x2, y2 = _xpose_tc(x, y, b, l)
    x2 = x2.reshape(l, d, b).transpose(2, 0, 1)
    y2 = y2.reshape(l, d, b).transpose(2, 0, 1)
    return x2, y2, loss


# speedup vs baseline: 1.4805x; 1.4805x over previous
import functools

import jax
import jax.numpy as jnp
from jax import lax
from jax.experimental import pallas as pl
from jax.experimental.pallas import tpu as pltpu
from jax.experimental.pallas import tpu_sc as plsc

_L2 = 1e-05
_LANES = 16
_NC, _NS = 2, 16
_NW = _NC * _NS
_CHUNK = 128


def _concat_tc(a, bb, rows_per_block):
    v, d = a.shape

    def body(a_ref, b_ref, o_ref):
        o_ref[...] = jnp.concatenate((a_ref[...], b_ref[...]), axis=-1)

    return pl.pallas_call(
        body,
        grid=(v // rows_per_block,),
        in_specs=[
            pl.BlockSpec((rows_per_block, d), lambda i: (i, 0)),
            pl.BlockSpec((rows_per_block, d), lambda i: (i, 0)),
        ],
        out_specs=pl.BlockSpec((rows_per_block, 2 * d), lambda i: (i, 0)),
        out_shape=jax.ShapeDtypeStruct((v, 2 * d), jnp.float32),
    )(a, bb)


def _combine_sc(qf, qrf, pidf, qq, rr, dflat):
    n = qf.shape[0]
    d2 = qq.shape[1]
    d = d2 // 2
    per_w = n // _NW
    steps = per_w // _CHUNK
    nchunk = d // _LANES
    assert steps % 2 == 0
    mesh = plsc.VectorSubcoreMesh(core_axis_name="c", subcore_axis_name="s")

    @functools.partial(
        pl.kernel,
        mesh=mesh,
        compiler_params=pltpu.CompilerParams(needs_layout_passes=False),
        out_type=[
            jax.ShapeDtypeStruct((n, d), jnp.float32),
            jax.ShapeDtypeStruct((n, d), jnp.float32),
            jax.ShapeDtypeStruct((_NW, _LANES), jnp.float32),
        ],
        scratch_types=[
            pltpu.VMEM((2, _CHUNK), jnp.int32),
            pltpu.VMEM((2, _CHUNK), jnp.int32),
            pltpu.VMEM((2, _CHUNK), jnp.int32),
            pltpu.VMEM((2, _CHUNK, d2), jnp.float32),
            pltpu.VMEM((2, _CHUNK, d2), jnp.float32),
            pltpu.VMEM((2, _CHUNK), jnp.float32),
            pltpu.VMEM((_CHUNK, d), jnp.float32),
            pltpu.VMEM((_CHUNK, d), jnp.float32),
            pltpu.VMEM((_LANES,), jnp.float32),
            pltpu.SemaphoreType.DMA,
            pltpu.SemaphoreType.DMA,
            pltpu.SemaphoreType.DMA,
        ],
    )
    def k(qf_hbm, qrf_hbm, pidf_hbm, qq_t, rr_t, df_t,
          x_hbm, y_hbm, part_hbm,
          iq_v, iqr_v, ipid_v, tq_v, tr_v, dc_v, xo_v, yo_v, acc_v,
          gsem0, gsem1, ssem):
        wid = lax.axis_index("s") * _NC + lax.axis_index("c")
        gsem = (gsem0, gsem1)
        acc_v[...] = jnp.zeros((_LANES,), jnp.float32)

        def fire_pair_idx(pair):
            gp = wid * (steps // 2) + pair
            pltpu.sync_copy(qf_hbm.at[gp], iq_v)
            pltpu.sync_copy(qrf_hbm.at[gp], iqr_v)
            pltpu.sync_copy(pidf_hbm.at[gp], ipid_v)

        def fire_g(b):
            pltpu.async_copy(qq_t.at[iq_v.at[b]], tq_v.at[b], gsem[b])
            pltpu.async_copy(rr_t.at[iqr_v.at[b]], tr_v.at[b], gsem[b])
            pltpu.async_copy(df_t.at[ipid_v.at[b]], dc_v.at[b], gsem[b])

        def wait_g(b):
            pltpu.make_async_copy(qq_t.at[iq_v.at[b]], tq_v.at[b], gsem[b]).wait()
            pltpu.make_async_copy(rr_t.at[iqr_v.at[b]], tr_v.at[b], gsem[b]).wait()
            pltpu.make_async_copy(df_t.at[ipid_v.at[b]], dc_v.at[b], gsem[b]).wait()

        def compute(b):
            @pl.loop(0, _CHUNK // _LANES)
            def _(g):
                dg = dc_v[b, pl.ds(g * _LANES, _LANES)]
                acc_v[...] += dg * dg

            @pl.loop(0, _CHUNK)
            def _(i):
                dv = plsc.load_gather(
                    dc_v.at[b], [jnp.full((_LANES,), i, jnp.int32)])
                for cc in range(nchunk):
                    sl = pl.ds(cc * _LANES, _LANES)
                    sh = pl.ds(d + cc * _LANES, _LANES)
                    xo_v[i, sl] = tq_v[b, i, sl] + dv * tq_v[b, i, sh]
                    yo_v[i, sl] = tr_v[b, i, sl] + dv * tr_v[b, i, sh]

        def store(step):
            base = wid * per_w + step * _CHUNK
            pltpu.async_copy(xo_v, x_hbm.at[pl.ds(base, _CHUNK)], ssem)
            pltpu.async_copy(yo_v, y_hbm.at[pl.ds(base, _CHUNK)], ssem)

        def wait_s():
            pltpu.make_async_copy(xo_v, x_hbm.at[pl.ds(0, _CHUNK)], ssem).wait()
            pltpu.make_async_copy(yo_v, y_hbm.at[pl.ds(0, _CHUNK)], ssem).wait()

        fire_pair_idx(0)
        fire_g(0)
        fire_g(1)
        wait_g(0)
        compute(0)
        store(0)

        @pl.loop(0, (steps - 2) // 2)
        def _(p):
            s1 = 2 * p + 1
            s2 = s1 + 1
            wait_g(1)
            fire_pair_idx(p + 1)
            fire_g(0)
            wait_s()
            compute(1)
            store(s1)
            fire_g(1)
            wait_g(0)
            wait_s()
            compute(0)
            store(s2)

        wait_g(1)
        wait_s()
        compute(1)
        store(steps - 1)
        wait_s()
        pltpu.sync_copy(acc_v, part_hbm.at[wid])

    shp = (n // (2 * _CHUNK), 2, _CHUNK)
    return k(qf.reshape(shp), qrf.reshape(shp), pidf.reshape(shp),
             qq, rr, dflat)


def _loss_tc(partials):
    def body(p_ref, o_ref):
        o_ref[0, 0] = jnp.sum(p_ref[...]) * jnp.float32(_L2)

    return pl.pallas_call(
        body,
        out_shape=jax.ShapeDtypeStruct((1, 1), jnp.float32),
        out_specs=pl.BlockSpec(memory_space=pltpu.SMEM),
    )(partials)


def kernel(q, qr, pid, q_emb, q_emb_diff, qr_emb, qr_emb_diff, diff_emb):
    b, l = q.shape
    d = q_emb.shape[1]
    qf = q.reshape(-1).astype(jnp.int32)
    qrf = qr.reshape(-1).astype(jnp.int32)
    pidf = pid.reshape(-1).astype(jnp.int32)
    qq = jnp.concatenate([q_emb, q_emb_diff], axis=1)
    rr = jnp.concatenate([qr_emb, qr_emb_diff], axis=1)
    dflat = diff_emb.reshape(-1)
    x, y, parts = _combine_sc(qf, qrf, pidf, qq, rr, dflat)
    loss = _loss_tc(parts)[0, 0]
    return x.reshape(b, l, d), y.reshape(b, l, d), loss
